# baseline (speedup 1.0000x reference)
import jax
import jax.numpy as jnp
from jax.experimental import pallas as pl

BLK_S = 1024
BLK_B = 2


def _add_kernel(x_ref, pos_ref, o_ref):
    o_ref[...] = x_ref[...] + pos_ref[...]


def kernel(x, pos_table):
    batch, seq_len, embed = x.shape
    grid = (seq_len // BLK_S, batch // BLK_B)
    return pl.pallas_call(
        _add_kernel,
        grid=grid,
        in_specs=[
            pl.BlockSpec((BLK_B, BLK_S, embed), lambda s, b: (b, s, 0)),
            pl.BlockSpec((BLK_S, embed), lambda s, b: (s, 0)),
        ],
        out_specs=pl.BlockSpec((BLK_B, BLK_S, embed), lambda s, b: (b, s, 0)),
        out_shape=jax.ShapeDtypeStruct((batch, seq_len, embed), x.dtype),
    )(x, pos_table[:seq_len])
